# baseline (speedup 1.0000x reference)
import jax
import jax.numpy as jnp
from jax import lax
from jax.experimental import pallas as pl
from jax.experimental.pallas import tpu as pltpu
from jax.experimental.pallas import tpu_sc as plsc

N_I = 16384
N_J = 50
DIM = 64
IB = 128
KB = DIM // 8
NCH = N_J * (N_I // IB)
NC, NS = 2, 16
NW = NC * NS
PER_W = NCH // NW
NBUF = 6
OUT_ROWS = NCH * DIM
OPAD = 129


def _body(idx_hbm, table_hbm, out_hbm, idx_v, rows_v, outv, gsem, wsem):
    wid = lax.axis_index("s") * NC + lax.axis_index("c")

    pltpu.sync_copy(idx_hbm.at[wid], idx_v)

    lanes = lax.iota(jnp.int32, 16)
    k_sets = [k0 * 16 + lanes for k0 in range(4)]

    def gather(t, b):
        pltpu.async_copy(table_hbm.at[idx_v.at[t]], rows_v.at[b],
                         gsem.at[b])

    def wait_gather(b):
        pltpu.make_async_copy(
            table_hbm.at[pl.ds(0, IB)], rows_v.at[b], gsem.at[b]).wait()

    def transpose(b):
        @pl.loop(0, IB, unroll=4)
        def _row(di):
            dv = jnp.full((16,), 0, jnp.int32) + di
            for g in range(4):
                v = rows_v[b, di, pl.ds(g * 16, 16)]
                plsc.store_scatter(outv.at[b], [k_sets[g], dv], v)

    def write(t, b):
        cg = wid * PER_W + t
        j = cg // IB
        ib = cg % IB
        for kb in range(KB):
            m0 = (j * KB + kb) * 1024 + ib * 8
            pltpu.async_copy(
                outv.at[b, pl.ds(kb * 8, 8), pl.ds(0, 128)],
                out_hbm.at[pl.ds(m0, 8)], wsem.at[b])

    def wait_write(b):
        for _ in range(KB):
            pltpu.make_async_copy(
                outv.at[b, pl.ds(0, 8), pl.ds(0, 128)],
                out_hbm.at[pl.ds(0, 8)], wsem.at[b]).wait()

    for b in range(NBUF):
        gather(b, b)

    @pl.loop(0, PER_W)
    def _visit(t):
        b = lax.rem(t, NBUF)
        wait_gather(b)

        @pl.when(t >= NBUF)
        def _():
            wait_write(b)

        transpose(b)

        @pl.when(t + NBUF < PER_W)
        def _():
            gather(t + NBUF, b)

        write(t, b)

    for b in range(NBUF):
        wait_write(b)


def kernel(data, ivectors_weight):
    idx = jnp.transpose(data.astype(jnp.int32)).reshape(NW, PER_W, IB)
    mesh = plsc.VectorSubcoreMesh(core_axis_name="c", subcore_axis_name="s")
    out_flat = pl.kernel(
        _body,
        out_type=jax.ShapeDtypeStruct((OUT_ROWS, 128), jnp.float32),
        mesh=mesh,
        scratch_types=[
            pltpu.VMEM((PER_W, IB), jnp.int32),
            pltpu.VMEM((NBUF, IB, DIM), jnp.float32),
            pltpu.VMEM((NBUF, DIM, OPAD), jnp.float32),
            pltpu.SemaphoreType.DMA((NBUF,)),
            pltpu.SemaphoreType.DMA((NBUF,)),
        ],
        compiler_params=pltpu.CompilerParams(
            use_tc_tiling_on_sc=False, needs_layout_passes=False),
    )(idx, ivectors_weight)
    out5 = out_flat.reshape(N_J, KB, IB, 8, 128)
    return out5.transpose(2, 4, 0, 1, 3).reshape(N_I, N_J, DIM)


# speedup vs baseline: 1.0580x; 1.0580x over previous
import jax
import jax.numpy as jnp
from jax import lax
from jax.experimental import pallas as pl
from jax.experimental.pallas import tpu as pltpu
from jax.experimental.pallas import tpu_sc as plsc

N_I = 16384
N_J = 50
DIM = 64
IB = 128
KB = DIM // 8
NCH = N_J * (N_I // IB)
NC, NS = 2, 16
NW = NC * NS
PER_W = NCH // NW
NBUF = 4
OUT_ROWS = NCH * DIM
OPAD = 129


def _body(idx_hbm, table_hbm, out_hbm, idx_v, rows_v, outv, gsem, wsem):
    wid = lax.axis_index("s") * NC + lax.axis_index("c")

    pltpu.sync_copy(idx_hbm.at[wid], idx_v)

    lanes = lax.iota(jnp.int32, 16)
    k_sets = [k0 * 16 + lanes for k0 in range(4)]

    def gather(t, b):
        pltpu.async_copy(table_hbm.at[idx_v.at[t]], rows_v.at[b],
                         gsem.at[b])

    def wait_gather(b):
        pltpu.make_async_copy(
            table_hbm.at[pl.ds(0, IB)], rows_v.at[b], gsem.at[b]).wait()

    def transpose(b):
        @pl.loop(0, IB, unroll=4)
        def _row(di):
            dv = jnp.full((16,), 0, jnp.int32) + di
            for g in range(4):
                v = rows_v[b, di, pl.ds(g * 16, 16)]
                plsc.store_scatter(outv.at[b], [k_sets[g], dv], v)

    def write(t, b):
        cg = wid * PER_W + t
        j = cg // IB
        ib = cg % IB
        for kb in range(KB):
            m0 = (j * KB + kb) * 1024 + ib * 8
            pltpu.async_copy(
                outv.at[b, pl.ds(kb * 8, 8), pl.ds(0, 128)],
                out_hbm.at[pl.ds(m0, 8)], wsem.at[b])

    def wait_write(b):
        for _ in range(KB):
            pltpu.make_async_copy(
                outv.at[b, pl.ds(0, 8), pl.ds(0, 128)],
                out_hbm.at[pl.ds(0, 8)], wsem.at[b]).wait()

    for b in range(NBUF):
        gather(b, b)

    @pl.loop(0, PER_W)
    def _visit(t):
        b = lax.rem(t, NBUF)
        wait_gather(b)

        @pl.when(t >= NBUF)
        def _():
            wait_write(b)

        transpose(b)

        @pl.when(t + NBUF < PER_W)
        def _():
            gather(t + NBUF, b)

        write(t, b)

    for b in range(NBUF):
        wait_write(b)


def kernel(data, ivectors_weight):
    idx = jnp.transpose(data.astype(jnp.int32)).reshape(NW, PER_W, IB)
    mesh = plsc.VectorSubcoreMesh(core_axis_name="c", subcore_axis_name="s")
    out_flat = pl.kernel(
        _body,
        out_type=jax.ShapeDtypeStruct((OUT_ROWS, 128), jnp.float32),
        mesh=mesh,
        scratch_types=[
            pltpu.VMEM((PER_W, IB), jnp.int32),
            pltpu.VMEM((NBUF, IB, DIM), jnp.float32),
            pltpu.VMEM((NBUF, DIM, OPAD), jnp.float32),
            pltpu.SemaphoreType.DMA((NBUF,)),
            pltpu.SemaphoreType.DMA((NBUF,)),
        ],
        compiler_params=pltpu.CompilerParams(
            use_tc_tiling_on_sc=False, needs_layout_passes=False),
    )(idx, ivectors_weight)
    out5 = out_flat.reshape(N_J, KB, IB, 8, 128)
    return out5.transpose(2, 4, 0, 1, 3).reshape(N_I, N_J, DIM)


# speedup vs baseline: 1.0613x; 1.0032x over previous
import jax
import jax.numpy as jnp
from jax import lax
from jax.experimental import pallas as pl
from jax.experimental.pallas import tpu as pltpu
from jax.experimental.pallas import tpu_sc as plsc

N_I = 16384
N_J = 50
DIM = 64
IB = 128
KB = DIM // 8
NCH = N_J * (N_I // IB)
NC, NS = 2, 16
NW = NC * NS
PER_W = NCH // NW
NBUF = 4
OUT_ROWS = NCH * DIM
OPAD = 129


def _body(idx_hbm, table_hbm, out_hbm, idx_v, rows_v, outv, gsem, wsem):
    wid = lax.axis_index("s") * NC + lax.axis_index("c")

    pltpu.sync_copy(idx_hbm.at[wid], idx_v)

    lanes = lax.iota(jnp.int32, 16)
    k_sets = [k0 * 16 + lanes for k0 in range(4)]

    def gather(t, b):
        pltpu.async_copy(table_hbm.at[idx_v.at[t]], rows_v.at[b],
                         gsem.at[b])

    def wait_gather(b):
        pltpu.make_async_copy(
            table_hbm.at[pl.ds(0, IB)], rows_v.at[b], gsem.at[b]).wait()

    def transpose(b):
        @pl.loop(0, IB, unroll=4)
        def _row(di):
            dv = jnp.full((16,), 0, jnp.int32) + di
            for g in range(4):
                v = rows_v[b, di, pl.ds(g * 16, 16)]
                plsc.store_scatter(outv.at[b], [k_sets[g], dv], v)

    def write(t, b):
        cg = wid * PER_W + t
        j = cg // IB
        ib = cg % IB
        for kb in range(KB):
            m0 = (j * KB + kb) * 1024 + ib * 8
            pltpu.async_copy(
                outv.at[b, pl.ds(kb * 8, 8), pl.ds(0, 128)],
                out_hbm.at[pl.ds(m0, 8)], wsem.at[b])

    def wait_write(b):
        pltpu.make_async_copy(
            outv.at[b, :, pl.ds(0, 128)],
            out_hbm.at[pl.ds(0, 64)], wsem.at[b]).wait()

    for b in range(NBUF):
        gather(b, b)

    @pl.loop(0, PER_W)
    def _visit(t):
        b = lax.rem(t, NBUF)
        wait_gather(b)

        @pl.when(t >= NBUF)
        def _():
            wait_write(b)

        transpose(b)

        @pl.when(t + NBUF < PER_W)
        def _():
            gather(t + NBUF, b)

        write(t, b)

    for b in range(NBUF):
        wait_write(b)


def kernel(data, ivectors_weight):
    idx = jnp.transpose(data.astype(jnp.int32)).reshape(NW, PER_W, IB)
    mesh = plsc.VectorSubcoreMesh(core_axis_name="c", subcore_axis_name="s")
    out_flat = pl.kernel(
        _body,
        out_type=jax.ShapeDtypeStruct((OUT_ROWS, 128), jnp.float32),
        mesh=mesh,
        scratch_types=[
            pltpu.VMEM((PER_W, IB), jnp.int32),
            pltpu.VMEM((NBUF, IB, DIM), jnp.float32),
            pltpu.VMEM((NBUF, DIM, OPAD), jnp.float32),
            pltpu.SemaphoreType.DMA((NBUF,)),
            pltpu.SemaphoreType.DMA((NBUF,)),
        ],
        compiler_params=pltpu.CompilerParams(
            use_tc_tiling_on_sc=False, needs_layout_passes=False),
    )(idx, ivectors_weight)
    out5 = out_flat.reshape(N_J, KB, IB, 8, 128)
    return out5.transpose(2, 4, 0, 1, 3).reshape(N_I, N_J, DIM)


# speedup vs baseline: 1.0751x; 1.0130x over previous
import jax
import jax.numpy as jnp
from jax import lax
from jax.experimental import pallas as pl
from jax.experimental.pallas import tpu as pltpu
from jax.experimental.pallas import tpu_sc as plsc

N_I = 16384
N_J = 50
DIM = 64
IB = 128
KB = DIM // 8
NCH = N_J * (N_I // IB)
NC, NS = 2, 16
NW = NC * NS
PER_W = NCH // NW
NBUF = 4
OUT_ROWS = NCH * DIM
OPAD = 129


def _body(idx_hbm, table_hbm, out_hbm, idx_v, rows_v, outv, gsem, wsem):
    wid = lax.axis_index("s") * NC + lax.axis_index("c")

    pltpu.sync_copy(idx_hbm.at[wid], idx_v)

    lanes = lax.iota(jnp.int32, 16)
    k_sets = [k0 * 16 + lanes for k0 in range(4)]

    def gather(t, b):
        pltpu.async_copy(table_hbm.at[idx_v.at[t]], rows_v.at[b],
                         gsem.at[b])

    def wait_gather(b):
        pltpu.make_async_copy(
            table_hbm.at[pl.ds(0, IB)], rows_v.at[b], gsem.at[b]).wait()

    def transpose(b):
        @pl.loop(0, IB, unroll=8)
        def _row(di):
            dv = jnp.full((16,), 0, jnp.int32) + di
            for g in range(4):
                v = rows_v[b, di, pl.ds(g * 16, 16)]
                plsc.store_scatter(outv.at[b], [k_sets[g], dv], v)

    def write(t, b):
        cg = wid * PER_W + t
        j = cg // IB
        ib = cg % IB
        for kb in range(KB):
            m0 = (j * KB + kb) * 1024 + ib * 8
            pltpu.async_copy(
                outv.at[b, pl.ds(kb * 8, 8), pl.ds(0, 128)],
                out_hbm.at[pl.ds(m0, 8)], wsem.at[b])

    def wait_write(b):
        pltpu.make_async_copy(
            outv.at[b, :, pl.ds(0, 128)],
            out_hbm.at[pl.ds(0, 64)], wsem.at[b]).wait()

    for b in range(NBUF):
        gather(b, b)

    @pl.loop(0, PER_W)
    def _visit(t):
        b = lax.rem(t, NBUF)
        wait_gather(b)

        @pl.when(t >= NBUF)
        def _():
            wait_write(b)

        transpose(b)

        @pl.when(t + NBUF < PER_W)
        def _():
            gather(t + NBUF, b)

        write(t, b)

    for b in range(NBUF):
        wait_write(b)


def kernel(data, ivectors_weight):
    idx = jnp.transpose(data.astype(jnp.int32)).reshape(NW, PER_W, IB)
    mesh = plsc.VectorSubcoreMesh(core_axis_name="c", subcore_axis_name="s")
    out_flat = pl.kernel(
        _body,
        out_type=jax.ShapeDtypeStruct((OUT_ROWS, 128), jnp.float32),
        mesh=mesh,
        scratch_types=[
            pltpu.VMEM((PER_W, IB), jnp.int32),
            pltpu.VMEM((NBUF, IB, DIM), jnp.float32),
            pltpu.VMEM((NBUF, DIM, OPAD), jnp.float32),
            pltpu.SemaphoreType.DMA((NBUF,)),
            pltpu.SemaphoreType.DMA((NBUF,)),
        ],
        compiler_params=pltpu.CompilerParams(
            use_tc_tiling_on_sc=False, needs_layout_passes=False),
    )(idx, ivectors_weight)
    out5 = out_flat.reshape(N_J, KB, IB, 8, 128)
    return out5.transpose(2, 4, 0, 1, 3).reshape(N_I, N_J, DIM)
